# baseline (speedup 1.0000x reference)
import jax
import jax.numpy as jnp
from jax import lax
from jax.experimental import pallas as pl
from jax.experimental.pallas import tpu as pltpu
from jax.experimental.pallas import tpu_sc as plsc

N = 10000
NP = 10112
E = 320000
D = 128
H = 8
DH = 16

NC = 2
NS = 16
NW = NC * NS
EPW = 10000
CHUNK = 64
KCH = 158
KCH_A = KCH + 1
EPW_PAD = KCH * CHUNK
ROWS_PW = NP // NS


def _dense_first_body(x_ref, w_ref, m_ref, h_ref, elr_ref):
    h = jnp.dot(x_ref[...], w_ref[...], preferred_element_type=jnp.float32)
    h_ref[...] = h
    elr_ref[...] = jnp.dot(h, m_ref[...], preferred_element_type=jnp.float32)


def _finalize(o_ref, d_ref, k_ref, b_ref):
    o = o_ref[0] + o_ref[1]
    d = d_ref[0, :, :H] + d_ref[1, :, :H]
    dexp = jnp.dot(d, k_ref[...], preferred_element_type=jnp.float32)
    x = o / jnp.maximum(dexp, 1e-9) + b_ref[...]
    return jnp.where(x > 0, x, jnp.exp(jnp.minimum(x, 0.0)) - 1.0)


def _dense_mid_body(o_ref, d_ref, k_ref, b_ref, w_ref, m_ref, h_ref, elr_ref):
    x = _finalize(o_ref, d_ref, k_ref, b_ref)
    h = jnp.dot(x, w_ref[...], preferred_element_type=jnp.float32)
    h_ref[...] = h
    elr_ref[...] = jnp.dot(h, m_ref[...], preferred_element_type=jnp.float32)


def _dense_last_body(o_ref, d_ref, k_ref, b_ref, y_ref):
    y_ref[...] = _finalize(o_ref, d_ref, k_ref, b_ref)


_BN = 1264


def _tc_first(x, w, m):
    return pl.pallas_call(
        _dense_first_body,
        grid=(NP // _BN,),
        in_specs=[
            pl.BlockSpec((_BN, D), lambda i: (i, 0)),
            pl.BlockSpec((D, D), lambda i: (0, 0)),
            pl.BlockSpec((D, 16), lambda i: (0, 0)),
        ],
        out_specs=[
            pl.BlockSpec((_BN, D), lambda i: (i, 0)),
            pl.BlockSpec((_BN, 16), lambda i: (i, 0)),
        ],
        out_shape=[
            jax.ShapeDtypeStruct((NP, D), jnp.float32),
            jax.ShapeDtypeStruct((NP, 16), jnp.float32),
        ],
    )(x, w, m)


def _tc_mid(o, d, k, b, w, m):
    return pl.pallas_call(
        _dense_mid_body,
        grid=(NP // _BN,),
        in_specs=[
            pl.BlockSpec((NC, _BN, D), lambda i: (0, i, 0)),
            pl.BlockSpec((NC, _BN, 16), lambda i: (0, i, 0)),
            pl.BlockSpec((H, D), lambda i: (0, 0)),
            pl.BlockSpec((1, D), lambda i: (0, 0)),
            pl.BlockSpec((D, D), lambda i: (0, 0)),
            pl.BlockSpec((D, 16), lambda i: (0, 0)),
        ],
        out_specs=[
            pl.BlockSpec((_BN, D), lambda i: (i, 0)),
            pl.BlockSpec((_BN, 16), lambda i: (i, 0)),
        ],
        out_shape=[
            jax.ShapeDtypeStruct((NP, D), jnp.float32),
            jax.ShapeDtypeStruct((NP, 16), jnp.float32),
        ],
    )(o, d, k, b, w, m)


def _tc_last(o, d, k, b):
    return pl.pallas_call(
        _dense_last_body,
        grid=(NP // _BN,),
        in_specs=[
            pl.BlockSpec((NC, _BN, D), lambda i: (0, i, 0)),
            pl.BlockSpec((NC, _BN, 16), lambda i: (0, i, 0)),
            pl.BlockSpec((H, D), lambda i: (0, 0)),
            pl.BlockSpec((1, D), lambda i: (0, 0)),
        ],
        out_specs=pl.BlockSpec((_BN, D), lambda i: (i, 0)),
        out_shape=jax.ShapeDtypeStruct((NP, D), jnp.float32),
    )(o, d, k, b)


def _edge_body(h_hbm, elr_hbm, src_hbm, dst_hbm, z128_hbm, z16_hbm,
               o_hbm, den_hbm,
               idx_s, idx_d, g1, g2, hrows, idx_sc, ex, msg,
               sem_g0, sem_g1, sem_s0, sem_s1,
               acc_sh, den_sh):
    sems_g = (sem_g0, sem_g1)
    sems_s = (sem_s0, sem_s1)
    cid = lax.axis_index("c")
    sid = lax.axis_index("s")
    wid = cid * NS + sid
    row0 = sid * ROWS_PW

    pltpu.sync_copy(z128_hbm.at[pl.ds(row0, ROWS_PW)],
                    acc_sh.at[pl.ds(row0, ROWS_PW)])
    pltpu.sync_copy(z16_hbm.at[pl.ds(row0, ROWS_PW)],
                    den_sh.at[pl.ds(row0, ROWS_PW)])
    plsc.subcore_barrier()

    def fire_g(b, k):
        pltpu.sync_copy(src_hbm.at[wid, k], idx_s.at[b])
        pltpu.sync_copy(dst_hbm.at[wid, k], idx_d.at[b])
        pltpu.async_copy(elr_hbm.at[idx_s.at[b]], g1.at[b], sems_g[b])
        pltpu.async_copy(elr_hbm.at[idx_d.at[b]], g2.at[b], sems_g[b])
        pltpu.async_copy(h_hbm.at[idx_s.at[b]], hrows.at[b], sems_g[b])

    def wait_g(b):
        s = sems_g[b]
        pltpu.make_async_copy(elr_hbm.at[idx_s.at[b]], g1.at[b], s).wait()
        pltpu.make_async_copy(elr_hbm.at[idx_d.at[b]], g2.at[b], s).wait()
        pltpu.make_async_copy(h_hbm.at[idx_s.at[b]], hrows.at[b], s).wait()

    def fire_sc(b):
        s = sems_s[b]
        pltpu.async_copy(ex.at[b], den_sh.at[idx_sc.at[b]], s, add=True)
        pltpu.async_copy(msg.at[b], acc_sh.at[idx_sc.at[b]], s, add=True)

    def wait_sc(b):
        s = sems_s[b]
        pltpu.make_async_copy(ex.at[b], den_sh.at[idx_sc.at[b]], s).wait()
        pltpu.make_async_copy(msg.at[b], acc_sh.at[idx_sc.at[b]], s).wait()

    def compute(b, k):
        def exp_body(i, c2):
            e = g1[b, i, :] + lax.rev(g2[b, i, :], (0,))
            e = jnp.maximum(e, 0.2 * e)
            ex[b, i, :] = jnp.exp(e)
            return c2

        lax.fori_loop(0, CHUNK, exp_body, 0, unroll=8)

        def mul_body(i, c2):
            exv = ex[b, i, :]
            for hh in range(H):
                s = exv[hh]
                msg[b, i, pl.ds(hh * DH, DH)] = (
                    hrows[b, i, pl.ds(hh * DH, DH)] * s)
            return c2

        lax.fori_loop(0, CHUNK, mul_body, 0, unroll=2)
        for v in range(CHUNK // 16):
            idx_sc[b, pl.ds(v * 16, 16)] = idx_d[b, pl.ds(v * 16, 16)]

    def step(b, k, warm):
        fire_g(1 - b, k + 1)
        wait_g(b)
        if warm:
            wait_sc(b)
        compute(b, k)
        fire_sc(b)

    fire_g(0, 0)
    step(0, 0, False)
    step(1, 1, False)

    def pipe_body(j, carry):
        step(0, 2 * j, True)
        step(1, 2 * j + 1, True)
        return carry

    lax.fori_loop(1, KCH // 2, pipe_body, 0)
    wait_g(0)
    wait_sc(0)
    wait_sc(1)
    plsc.subcore_barrier()

    pltpu.sync_copy(acc_sh.at[pl.ds(row0, ROWS_PW)],
                    o_hbm.at[cid, pl.ds(row0, ROWS_PW)])
    pltpu.sync_copy(den_sh.at[pl.ds(row0, ROWS_PW)],
                    den_hbm.at[cid, pl.ds(row0, ROWS_PW)])


def _sc_edge(h, elr, srcg, dstg, z128, z16):
    mesh = plsc.VectorSubcoreMesh(core_axis_name="c", subcore_axis_name="s")
    f = pl.kernel(
        _edge_body,
        out_type=[
            jax.ShapeDtypeStruct((NC, NP, D), jnp.float32),
            jax.ShapeDtypeStruct((NC, NP, 16), jnp.float32),
        ],
        mesh=mesh,
        compiler_params=pltpu.CompilerParams(use_tc_tiling_on_sc=False),
        scratch_types=[
            pltpu.VMEM((2, CHUNK), jnp.int32),
            pltpu.VMEM((2, CHUNK), jnp.int32),
            pltpu.VMEM((2, CHUNK, 16), jnp.float32),
            pltpu.VMEM((2, CHUNK, 16), jnp.float32),
            pltpu.VMEM((2, CHUNK, D), jnp.float32),
            pltpu.VMEM((2, CHUNK), jnp.int32),
            pltpu.VMEM((2, CHUNK, 16), jnp.float32),
            pltpu.VMEM((2, CHUNK, D), jnp.float32),
            pltpu.SemaphoreType.DMA,
            pltpu.SemaphoreType.DMA,
            pltpu.SemaphoreType.DMA,
            pltpu.SemaphoreType.DMA,
            pltpu.VMEM_SHARED((NP, D), jnp.float32),
            pltpu.VMEM_SHARED((NP, 16), jnp.float32),
        ],
    )
    return f(h, elr, srcg, dstg, z128, z16)


def _pack_attn(al, ar):
    eye = jnp.eye(H, dtype=jnp.float32)
    al_m = (al[:, :, None] * eye[:, None, :]).reshape(D, H)
    ar_m = (ar[:, :, None] * eye[:, None, :]).reshape(D, H)
    return jnp.concatenate([al_m, ar_m[:, ::-1]], axis=1)


def kernel(features, edge_index, W1, al1, ar1, b1, W2, al2, ar2, b2):
    npad = KCH_A * CHUNK - EPW
    padv = N + (jnp.arange(npad, dtype=jnp.int32) % (NP - N))
    padv = jnp.broadcast_to(padv, (NW, npad))

    def _group(v):
        return jnp.concatenate([v.reshape(NW, EPW), padv],
                               axis=1).reshape(NW, KCH_A, CHUNK)

    srcg = _group(edge_index[0].astype(jnp.int32))
    dstg = _group(edge_index[1].astype(jnp.int32))

    m1 = _pack_attn(al1, ar1)
    m2 = _pack_attn(al2, ar2)
    kexp = (jnp.eye(H, dtype=jnp.float32)[:, :, None]
            * jnp.ones((DH,), jnp.float32)).reshape(H, D)
    z128 = jnp.zeros((NP, D), jnp.float32)
    z16 = jnp.zeros((NP, 16), jnp.float32)

    xpad = jnp.pad(features, ((0, NP - N), (0, 0)))
    h1, elr1 = _tc_first(xpad, W1, m1)
    o1, d1 = _sc_edge(h1, elr1, srcg, dstg, z128, z16)
    h2, elr2 = _tc_mid(o1, d1, kexp, b1.reshape(1, D), W2, m2)
    o2, d2 = _sc_edge(h2, elr2, srcg, dstg, z128, z16)
    return _tc_last(o2, d2, kexp, b2.reshape(1, D))[:N]


# speedup vs baseline: 2.5677x; 2.5677x over previous
import jax
import jax.numpy as jnp
from jax import lax
from jax.experimental import pallas as pl
from jax.experimental.pallas import tpu as pltpu
from jax.experimental.pallas import tpu_sc as plsc

N = 10000
NP = 10112
E = 320000
D = 128
H = 8
DH = 16

NC = 2
NS = 16
NW = NC * NS
EPW = 10000
CHUNK = 64
KCH = 158
KCH_A = KCH + 1
EPW_PAD = KCH * CHUNK
ROWS_PW = NP // NS


def _dense_first_body(x_ref, w_ref, m_ref, h_ref, elr_ref):
    h = jnp.dot(x_ref[...], w_ref[...], preferred_element_type=jnp.float32)
    h_ref[...] = h
    elr_ref[...] = jnp.dot(h, m_ref[...], preferred_element_type=jnp.float32)


def _finalize(o_ref, d_ref, k_ref, b_ref):
    o = o_ref[0] + o_ref[1]
    d = d_ref[0, :, :H] + d_ref[1, :, :H]
    dexp = jnp.dot(d, k_ref[...], preferred_element_type=jnp.float32)
    x = o / jnp.maximum(dexp, 1e-9) + b_ref[...]
    return jnp.where(x > 0, x, jnp.exp(jnp.minimum(x, 0.0)) - 1.0)


def _dense_mid_body(o_ref, d_ref, k_ref, b_ref, w_ref, m_ref, h_ref, elr_ref):
    x = _finalize(o_ref, d_ref, k_ref, b_ref)
    h = jnp.dot(x, w_ref[...], preferred_element_type=jnp.float32)
    h_ref[...] = h
    elr_ref[...] = jnp.dot(h, m_ref[...], preferred_element_type=jnp.float32)


def _dense_last_body(o_ref, d_ref, k_ref, b_ref, y_ref):
    y_ref[...] = _finalize(o_ref, d_ref, k_ref, b_ref)


_BN = 1264


def _tc_first(x, w, m):
    return pl.pallas_call(
        _dense_first_body,
        grid=(NP // _BN,),
        in_specs=[
            pl.BlockSpec((_BN, D), lambda i: (i, 0)),
            pl.BlockSpec((D, D), lambda i: (0, 0)),
            pl.BlockSpec((D, 16), lambda i: (0, 0)),
        ],
        out_specs=[
            pl.BlockSpec((_BN, D), lambda i: (i, 0)),
            pl.BlockSpec((_BN, 16), lambda i: (i, 0)),
        ],
        out_shape=[
            jax.ShapeDtypeStruct((NP, D), jnp.float32),
            jax.ShapeDtypeStruct((NP, 16), jnp.float32),
        ],
    )(x, w, m)


def _tc_mid(o, d, k, b, w, m):
    return pl.pallas_call(
        _dense_mid_body,
        grid=(NP // _BN,),
        in_specs=[
            pl.BlockSpec((NC, _BN, D), lambda i: (0, i, 0)),
            pl.BlockSpec((NC, _BN, 16), lambda i: (0, i, 0)),
            pl.BlockSpec((H, D), lambda i: (0, 0)),
            pl.BlockSpec((1, D), lambda i: (0, 0)),
            pl.BlockSpec((D, D), lambda i: (0, 0)),
            pl.BlockSpec((D, 16), lambda i: (0, 0)),
        ],
        out_specs=[
            pl.BlockSpec((_BN, D), lambda i: (i, 0)),
            pl.BlockSpec((_BN, 16), lambda i: (i, 0)),
        ],
        out_shape=[
            jax.ShapeDtypeStruct((NP, D), jnp.float32),
            jax.ShapeDtypeStruct((NP, 16), jnp.float32),
        ],
    )(o, d, k, b, w, m)


def _tc_last(o, d, k, b):
    return pl.pallas_call(
        _dense_last_body,
        grid=(NP // _BN,),
        in_specs=[
            pl.BlockSpec((NC, _BN, D), lambda i: (0, i, 0)),
            pl.BlockSpec((NC, _BN, 16), lambda i: (0, i, 0)),
            pl.BlockSpec((H, D), lambda i: (0, 0)),
            pl.BlockSpec((1, D), lambda i: (0, 0)),
        ],
        out_specs=pl.BlockSpec((_BN, D), lambda i: (i, 0)),
        out_shape=jax.ShapeDtypeStruct((NP, D), jnp.float32),
    )(o, d, k, b)


def _edge_body(h_hbm, elr_hbm, src_hbm, dst_hbm, z128_hbm, z16_hbm,
               o_hbm, den_hbm,
               idx_s, idx_d, g1, g2, hrows, idx_sc, ex, msg,
               sem_g0, sem_g1, sem_s0, sem_s1,
               acc_sh, den_sh):
    sems_g = (sem_g0, sem_g1)
    sems_s = (sem_s0, sem_s1)
    cid = lax.axis_index("c")
    sid = lax.axis_index("s")
    wid = cid * NS + sid
    row0 = sid * ROWS_PW

    pltpu.sync_copy(z128_hbm.at[pl.ds(row0, ROWS_PW)],
                    acc_sh.at[pl.ds(row0, ROWS_PW)])
    pltpu.sync_copy(z16_hbm.at[pl.ds(row0, ROWS_PW)],
                    den_sh.at[pl.ds(row0, ROWS_PW)])
    plsc.subcore_barrier()

    def fire_g(b, k):
        pltpu.sync_copy(src_hbm.at[wid, k], idx_s.at[b])
        pltpu.sync_copy(dst_hbm.at[wid, k], idx_d.at[b])
        pltpu.async_copy(elr_hbm.at[idx_s.at[b]], g1.at[b], sems_g[b])
        pltpu.async_copy(elr_hbm.at[idx_d.at[b]], g2.at[b], sems_g[b])
        pltpu.async_copy(h_hbm.at[idx_s.at[b]], hrows.at[b], sems_g[b])

    def wait_g(b):
        s = sems_g[b]
        pltpu.make_async_copy(elr_hbm.at[idx_s.at[b]], g1.at[b], s).wait()
        pltpu.make_async_copy(elr_hbm.at[idx_d.at[b]], g2.at[b], s).wait()
        pltpu.make_async_copy(h_hbm.at[idx_s.at[b]], hrows.at[b], s).wait()

    def fire_sc(b):
        s = sems_s[b]
        pltpu.async_copy(ex.at[b], den_sh.at[idx_sc.at[b]], s, add=True)
        pltpu.async_copy(msg.at[b], acc_sh.at[idx_sc.at[b]], s, add=True)

    def wait_sc(b):
        s = sems_s[b]
        pltpu.make_async_copy(ex.at[b], den_sh.at[idx_sc.at[b]], s).wait()
        pltpu.make_async_copy(msg.at[b], acc_sh.at[idx_sc.at[b]], s).wait()

    def compute(b, k):
        @plsc.parallel_loop(0, CHUNK, step=1, unroll=4)
        def _(i):
            e = g1[b, i, :] + lax.rev(g2[b, i, :], (0,))
            e = jnp.maximum(e, 0.2 * e)
            exv = jnp.exp(e)
            ex[b, i, :] = exv
            for hh in range(H):
                s = exv[hh]
                msg[b, i, pl.ds(hh * DH, DH)] = (
                    hrows[b, i, pl.ds(hh * DH, DH)] * s)
        for v in range(CHUNK // 16):
            idx_sc[b, pl.ds(v * 16, 16)] = idx_d[b, pl.ds(v * 16, 16)]

    def step(b, k, warm):
        fire_g(1 - b, k + 1)
        wait_g(b)
        if warm:
            wait_sc(b)
        compute(b, k)
        fire_sc(b)

    fire_g(0, 0)
    step(0, 0, False)
    step(1, 1, False)

    def pipe_body(j, carry):
        step(0, 2 * j, True)
        step(1, 2 * j + 1, True)
        return carry

    lax.fori_loop(1, KCH // 2, pipe_body, 0)
    wait_g(0)
    wait_sc(0)
    wait_sc(1)
    plsc.subcore_barrier()

    pltpu.sync_copy(acc_sh.at[pl.ds(row0, ROWS_PW)],
                    o_hbm.at[cid, pl.ds(row0, ROWS_PW)])
    pltpu.sync_copy(den_sh.at[pl.ds(row0, ROWS_PW)],
                    den_hbm.at[cid, pl.ds(row0, ROWS_PW)])


def _sc_edge(h, elr, srcg, dstg, z128, z16):
    mesh = plsc.VectorSubcoreMesh(core_axis_name="c", subcore_axis_name="s")
    f = pl.kernel(
        _edge_body,
        out_type=[
            jax.ShapeDtypeStruct((NC, NP, D), jnp.float32),
            jax.ShapeDtypeStruct((NC, NP, 16), jnp.float32),
        ],
        mesh=mesh,
        compiler_params=pltpu.CompilerParams(use_tc_tiling_on_sc=False),
        scratch_types=[
            pltpu.VMEM((2, CHUNK), jnp.int32),
            pltpu.VMEM((2, CHUNK), jnp.int32),
            pltpu.VMEM((2, CHUNK, 16), jnp.float32),
            pltpu.VMEM((2, CHUNK, 16), jnp.float32),
            pltpu.VMEM((2, CHUNK, D), jnp.float32),
            pltpu.VMEM((2, CHUNK), jnp.int32),
            pltpu.VMEM((2, CHUNK, 16), jnp.float32),
            pltpu.VMEM((2, CHUNK, D), jnp.float32),
            pltpu.SemaphoreType.DMA,
            pltpu.SemaphoreType.DMA,
            pltpu.SemaphoreType.DMA,
            pltpu.SemaphoreType.DMA,
            pltpu.VMEM_SHARED((NP, D), jnp.float32),
            pltpu.VMEM_SHARED((NP, 16), jnp.float32),
        ],
    )
    return f(h, elr, srcg, dstg, z128, z16)


def _pack_attn(al, ar):
    eye = jnp.eye(H, dtype=jnp.float32)
    al_m = (al[:, :, None] * eye[:, None, :]).reshape(D, H)
    ar_m = (ar[:, :, None] * eye[:, None, :]).reshape(D, H)
    return jnp.concatenate([al_m, ar_m[:, ::-1]], axis=1)


def kernel(features, edge_index, W1, al1, ar1, b1, W2, al2, ar2, b2):
    npad = KCH_A * CHUNK - EPW
    padv = N + (jnp.arange(npad, dtype=jnp.int32) % (NP - N))
    padv = jnp.broadcast_to(padv, (NW, npad))

    def _group(v):
        return jnp.concatenate([v.reshape(NW, EPW), padv],
                               axis=1).reshape(NW, KCH_A, CHUNK)

    srcg = _group(edge_index[0].astype(jnp.int32))
    dstg = _group(edge_index[1].astype(jnp.int32))

    m1 = _pack_attn(al1, ar1)
    m2 = _pack_attn(al2, ar2)
    kexp = (jnp.eye(H, dtype=jnp.float32)[:, :, None]
            * jnp.ones((DH,), jnp.float32)).reshape(H, D)
    z128 = jnp.zeros((NP, D), jnp.float32)
    z16 = jnp.zeros((NP, 16), jnp.float32)

    xpad = jnp.pad(features, ((0, NP - N), (0, 0)))
    h1, elr1 = _tc_first(xpad, W1, m1)
    o1, d1 = _sc_edge(h1, elr1, srcg, dstg, z128, z16)
    h2, elr2 = _tc_mid(o1, d1, kexp, b1.reshape(1, D), W2, m2)
    o2, d2 = _sc_edge(h2, elr2, srcg, dstg, z128, z16)
    return _tc_last(o2, d2, kexp, b2.reshape(1, D))[:N]
